# baseline (speedup 1.0000x reference)
import functools

import jax
import jax.numpy as jnp
from jax import lax
from jax.experimental import pallas as pl
from jax.experimental.pallas import tpu as pltpu
from jax.experimental.pallas import tpu_sc as plsc

HIDDEN = 768
MAX_POS = 77
N_ROWS = 4096 * 77
NC, NS, L = 2, 16, 16
NW = NC * NS
CHUNK = 64
CHUNKS_PER_W = N_ROWS // (NW * CHUNK)
VPR = HIDDEN // L

_mesh = plsc.VectorSubcoreMesh(core_axis_name="c", subcore_axis_name="s")


@functools.partial(
    pl.kernel,
    out_type=jax.ShapeDtypeStruct((N_ROWS, HIDDEN), jnp.float32),
    mesh=_mesh,
    scratch_types=[
        pltpu.VMEM((CHUNK,), jnp.int32),
        pltpu.VMEM((CHUNK,), jnp.int32),
        pltpu.VMEM((CHUNK, HIDDEN), jnp.float32),
        pltpu.VMEM((CHUNK, HIDDEN), jnp.float32),
        pltpu.SemaphoreType.DMA,
        pltpu.SemaphoreType.DMA,
    ],
)
def _emb_kernel(ids_hbm, tok_hbm, pos_hbm, out_hbm,
                idx_v, pidx_v, tok_b, pos_b, sem_g, sem_p):
  wid = lax.axis_index("s") * NC + lax.axis_index("c")
  w_base = wid * CHUNKS_PER_W * CHUNK

  def chunk_body(c, carry):
    row_base = w_base + c * CHUNK
    pltpu.sync_copy(ids_hbm.at[pl.ds(row_base, CHUNK)], idx_v)
    g = pltpu.async_copy(tok_hbm.at[idx_v], tok_b, sem_g)
    iota = lax.iota(jnp.int32, L)
    for k in range(CHUNK // L):
      pidx_v[pl.ds(k * L, L)] = (iota + (row_base + k * L)) % MAX_POS
    p = pltpu.async_copy(pos_hbm.at[pidx_v], pos_b, sem_p)
    g.wait()
    p.wait()

    def add_row(r, carry2):
      for k in range(VPR):
        sl = pl.ds(k * L, L)
        tok_b[r, sl] = tok_b[r, sl] + pos_b[r, sl]
      return carry2

    lax.fori_loop(0, CHUNK, add_row, 0)
    pltpu.sync_copy(tok_b, out_hbm.at[pl.ds(row_base, CHUNK)])
    return carry

  lax.fori_loop(0, CHUNKS_PER_W, chunk_body, 0)


def kernel(input_ids, token_table, pos_table):
  Bn, Tn = input_ids.shape
  ids = input_ids.reshape(-1).astype(jnp.int32)
  out = _emb_kernel(ids, token_table, pos_table)
  return out.reshape(Bn, Tn, HIDDEN)
